# Initial kernel scaffold; baseline (speedup 1.0000x reference)
import jax
import jax.numpy as jnp
from jax.experimental import pallas as pl


def kernel(x, pos, W_first, b_first, W_mid, b_mid, W_last, b_last):
    raise NotImplementedError("write your pallas kernel here")


# speedup vs baseline: 13.2372x; 13.2372x over previous
import functools

import jax
import jax.numpy as jnp
from jax import lax
from jax.experimental import pallas as pl
from jax.experimental.pallas import tpu as pltpu

B, N, D, KNN, GR = 4, 4096, 64, 16, 32
HD = 3 * GR
TILE = 256


def _hbase_body(x_ref, wf_ref, bf_ref, wm_ref, bm_ref, wl_ref, bl_ref,
                h_ref, base_ref):
    xb = x_ref[0]
    wf = wf_ref[...]
    wa, wb, wc = wf[0:D, :], wf[D:2 * D, :], wf[2 * D:3 * D, :]
    wm1, wm2 = wm_ref[0:GR, :], wm_ref[GR:GR + D, :]
    wl1, wl2, wl3 = wl_ref[0:GR, :], wl_ref[GR:2 * GR, :], wl_ref[2 * GR:, :]
    f32 = jnp.float32
    dot = functools.partial(jnp.dot, preferred_element_type=f32,
                            precision=lax.Precision.HIGHEST)
    u1 = wb + wc
    u2 = dot(u1, wm1)
    u3 = dot(u2, wl1) + dot(u1, wl2)
    a = dot(xb, wa - wc) + bf_ref[...]
    b = dot(a, wm1) + dot(xb, wm2) + bm_ref[...]
    c = dot(b, wl1) + dot(a, wl2) + dot(xb, wl3) + bl_ref[...]
    h_ref[0] = jnp.concatenate([dot(xb, u3), dot(xb, u2), dot(xb, u1)], axis=1)
    base_ref[0] = jnp.concatenate([c, b, a], axis=1)


def _main_body(post_ref, h_ref, base_ref, x_ref, out_ref):
    i = pl.program_id(1)
    rows = pl.ds(i * TILE, TILE)
    pos_t = post_ref[0]
    h = h_ref[0]
    f32 = jnp.float32

    d2_all = jnp.sum(pos_t * pos_t, axis=0)[None, :]
    pos_rows = post_ref[0, :, rows]
    cross = lax.dot_general(pos_rows.astype(jnp.bfloat16),
                            pos_t.astype(jnp.bfloat16),
                            (((0,), (0,)), ((), ())),
                            preferred_element_type=f32)
    d2_rows = jnp.sum(pos_rows * pos_rows, axis=0)[:, None]
    dist = d2_rows + d2_all - 2.0 * cross

    iota_col = lax.broadcasted_iota(jnp.int32, (TILE, N), 1)
    inf = jnp.float32(jnp.inf)

    def pick(dcur):
        rowmin = jnp.min(dcur, axis=1, keepdims=True)
        cand = jnp.where(dcur == rowmin, iota_col, N)
        idxm = jnp.min(cand, axis=1, keepdims=True)
        return iota_col == idxm

    onehot = pick(dist)
    dist = jnp.where(onehot, inf, dist)

    m = jnp.full((TILE, HD), -jnp.inf, dtype=f32)
    for _ in range(KNN):
        onehot = pick(dist)
        g = lax.dot_general(onehot.astype(f32), h,
                            (((1,), (0,)), ((), ())),
                            preferred_element_type=f32)
        m = jnp.maximum(m, g)
        dist = jnp.where(onehot, inf, dist)

    out_ref[0] = jnp.concatenate([base_ref[0] + m, x_ref[0]], axis=1)


def kernel(x, pos, W_first, b_first, W_mid, b_mid, W_last, b_last):
    f32 = jnp.float32
    b_first2 = b_first.reshape(1, GR)
    b_mid2 = b_mid.reshape(1, GR)
    b_last2 = b_last.reshape(1, GR)

    h, base = pl.pallas_call(
        _hbase_body,
        grid=(B,),
        in_specs=[
            pl.BlockSpec((1, N, D), lambda b: (b, 0, 0)),
            pl.BlockSpec((3 * D, GR), lambda b: (0, 0)),
            pl.BlockSpec((1, GR), lambda b: (0, 0)),
            pl.BlockSpec((D + GR, GR), lambda b: (0, 0)),
            pl.BlockSpec((1, GR), lambda b: (0, 0)),
            pl.BlockSpec((D + 2 * GR, GR), lambda b: (0, 0)),
            pl.BlockSpec((1, GR), lambda b: (0, 0)),
        ],
        out_specs=[
            pl.BlockSpec((1, N, HD), lambda b: (b, 0, 0)),
            pl.BlockSpec((1, N, HD), lambda b: (b, 0, 0)),
        ],
        out_shape=[
            jax.ShapeDtypeStruct((B, N, HD), f32),
            jax.ShapeDtypeStruct((B, N, HD), f32),
        ],
    )(x, W_first, b_first2, W_mid, b_mid2, W_last, b_last2)

    pos_t = jnp.concatenate(
        [pos.transpose(0, 2, 1), jnp.zeros((B, 5, N), f32)], axis=1)

    out = pl.pallas_call(
        _main_body,
        grid=(B, N // TILE),
        in_specs=[
            pl.BlockSpec((1, 8, N), lambda b, i: (b, 0, 0)),
            pl.BlockSpec((1, N, HD), lambda b, i: (b, 0, 0)),
            pl.BlockSpec((1, TILE, HD), lambda b, i: (b, i, 0)),
            pl.BlockSpec((1, TILE, D), lambda b, i: (b, i, 0)),
        ],
        out_specs=pl.BlockSpec((1, TILE, HD + D), lambda b, i: (b, i, 0)),
        out_shape=jax.ShapeDtypeStruct((B, N, HD + D), f32),
        compiler_params=pltpu.CompilerParams(
            dimension_semantics=("parallel", "arbitrary")),
    )(pos_t, h, base, x)
    return out
